# baseline (speedup 1.0000x reference)
import jax
import jax.numpy as jnp
from jax import lax
from jax.experimental import pallas as pl
from jax.experimental.pallas import tpu as pltpu
from jax.experimental.pallas import tpu_sc as plsc

NC = 2
NS = 16
NW = NC * NS
L = 16

B, T, D = 16, 2048, 512
B_SC = 4
B_TC = B - B_SC
ROWS_W = T // NW


def _sc_trim(seq_hbm, pe_hbm, out_hbm, pe_v, seq_v):
    wid = lax.axis_index("s") * NC + lax.axis_index("c")
    row0 = wid * ROWS_W

    pltpu.sync_copy(pe_hbm.at[0, pl.ds(row0, ROWS_W)], pe_v)

    @pl.loop(0, B_SC)
    def _batch(b):
        pltpu.sync_copy(seq_hbm.at[B_TC + b, pl.ds(row0, ROWS_W)], seq_v)

        @pl.loop(0, ROWS_W)
        def _row(r):
            @plsc.parallel_loop(0, D, step=L, unroll=8)
            def _col(c):
                seq_v[r, pl.ds(c, L)] = seq_v[r, pl.ds(c, L)] * 2.0 + pe_v[r, pl.ds(c, L)]

        pltpu.sync_copy(seq_v, out_hbm.at[b, 0, pl.ds(row0, ROWS_W)])


def _tc_trim(seq_ref, pe_ref, out_ref):
    out_ref[:, 0] = seq_ref[...] * 2.0 + pe_ref[...]


def kernel(seq, times, pe):
    del times
    pe2 = pe[0]

    out_tc = pl.pallas_call(
        _tc_trim,
        grid=(B_TC // 2,),
        in_specs=[
            pl.BlockSpec((2, T, D), lambda bi: (bi, 0, 0)),
            pl.BlockSpec((T, D), lambda bi: (0, 0)),
        ],
        out_specs=pl.BlockSpec((2, 1, T, D), lambda bi: (bi, 0, 0, 0)),
        out_shape=jax.ShapeDtypeStruct((B_TC, 1, T, D), seq.dtype),
    )(seq, pe2)

    mesh = plsc.VectorSubcoreMesh(core_axis_name="c", subcore_axis_name="s")
    out_sc = pl.kernel(
        _sc_trim,
        out_type=jax.ShapeDtypeStruct((B_SC, 1, T, D), jnp.float32),
        mesh=mesh,
        scratch_types=[
            pltpu.VMEM((ROWS_W, D), jnp.float32),
            pltpu.VMEM((ROWS_W, D), jnp.float32),
        ],
    )(seq, pe)

    out = jnp.concatenate([out_tc, out_sc], axis=0)
    mask = jnp.ones((B, 1), dtype=bool)
    return (out, mask)


# speedup vs baseline: 2.3362x; 2.3362x over previous
import jax
import jax.numpy as jnp
from jax.experimental import pallas as pl
from jax.experimental.pallas import tpu as pltpu

B, T, D = 16, 2048, 512
HALF = T // 2
K = B * 2
NB = 4


def _ring_body(seq_hbm, pe_hbm, out_hbm, pe_buf, in_bufs, out_bufs, pe_sem, in_sems, out_sems):
    def in_dma(i, slot):
        b, h = divmod(i, 2)
        return pltpu.make_async_copy(
            seq_hbm.at[b, pl.ds(h * HALF, HALF)], in_bufs.at[slot], in_sems.at[slot]
        )

    def out_dma(i, slot):
        b, h = divmod(i, 2)
        return pltpu.make_async_copy(
            out_bufs.at[slot], out_hbm.at[b, 0, pl.ds(h * HALF, HALF)], out_sems.at[slot]
        )

    pe_copy = pltpu.make_async_copy(pe_hbm.at[0], pe_buf, pe_sem)
    pe_copy.start()
    for i in range(NB):
        in_dma(i, i).start()
    pe_copy.wait()

    for i in range(K):
        slot = i % NB
        if i >= NB:
            out_dma(i - NB, slot).wait()
        in_dma(i, slot).wait()
        h = i % 2
        out_bufs[slot] = in_bufs[slot] * 2.0 + pe_buf[pl.ds(h * HALF, HALF)]
        out_dma(i, slot).start()
        if i + NB < K:
            in_dma(i + NB, slot).start()

    for i in range(K - NB, K):
        out_dma(i, i % NB).wait()


def kernel(seq, times, pe):
    del times
    out = pl.pallas_call(
        _ring_body,
        in_specs=[
            pl.BlockSpec(memory_space=pl.ANY),
            pl.BlockSpec(memory_space=pl.ANY),
        ],
        out_specs=pl.BlockSpec(memory_space=pl.ANY),
        out_shape=jax.ShapeDtypeStruct((B, 1, T, D), seq.dtype),
        scratch_shapes=[
            pltpu.VMEM((T, D), jnp.float32),
            pltpu.VMEM((NB, HALF, D), jnp.float32),
            pltpu.VMEM((NB, HALF, D), jnp.float32),
            pltpu.SemaphoreType.DMA,
            pltpu.SemaphoreType.DMA((NB,)),
            pltpu.SemaphoreType.DMA((NB,)),
        ],
    )(seq, pe)
    mask = jnp.ones((B, 1), dtype=bool)
    return (out, mask)


# speedup vs baseline: 2.3781x; 1.0179x over previous
import jax
import jax.numpy as jnp
from jax.experimental import pallas as pl
from jax.experimental.pallas import tpu as pltpu

B, T, D = 16, 2048, 512
HALF = T
K = B
NB = 3


def _ring_body(seq_hbm, pe_hbm, out_hbm, pe_buf, in_bufs, out_bufs, pe_sem, in_sems, out_sems):
    def in_dma(i, slot):
        b, h = i, 0
        return pltpu.make_async_copy(
            seq_hbm.at[b, pl.ds(h * HALF, HALF)], in_bufs.at[slot], in_sems.at[slot]
        )

    def out_dma(i, slot):
        b, h = i, 0
        return pltpu.make_async_copy(
            out_bufs.at[slot], out_hbm.at[b, 0, pl.ds(h * HALF, HALF)], out_sems.at[slot]
        )

    pe_copy = pltpu.make_async_copy(pe_hbm.at[0], pe_buf, pe_sem)
    pe_copy.start()
    for i in range(NB):
        in_dma(i, i).start()
    pe_copy.wait()

    for i in range(K):
        slot = i % NB
        if i >= NB:
            out_dma(i - NB, slot).wait()
        in_dma(i, slot).wait()
        h = 0
        out_bufs[slot] = in_bufs[slot] * 2.0 + pe_buf[pl.ds(h * HALF, HALF)]
        out_dma(i, slot).start()
        if i + NB < K:
            in_dma(i + NB, slot).start()

    for i in range(K - NB, K):
        out_dma(i, i % NB).wait()


def kernel(seq, times, pe):
    del times
    out = pl.pallas_call(
        _ring_body,
        in_specs=[
            pl.BlockSpec(memory_space=pl.ANY),
            pl.BlockSpec(memory_space=pl.ANY),
        ],
        out_specs=pl.BlockSpec(memory_space=pl.ANY),
        out_shape=jax.ShapeDtypeStruct((B, 1, T, D), seq.dtype),
        scratch_shapes=[
            pltpu.VMEM((T, D), jnp.float32),
            pltpu.VMEM((NB, HALF, D), jnp.float32),
            pltpu.VMEM((NB, HALF, D), jnp.float32),
            pltpu.SemaphoreType.DMA,
            pltpu.SemaphoreType.DMA((NB,)),
            pltpu.SemaphoreType.DMA((NB,)),
        ],
    )(seq, pe)
    mask = jnp.ones((B, 1), dtype=bool)
    return (out, mask)
